# baseline (speedup 1.0000x reference)
import functools

import jax
import jax.numpy as jnp
from jax import lax
from jax.experimental import pallas as pl
from jax.experimental.pallas import tpu as pltpu
from jax.experimental.pallas import tpu_sc as plsc

N = 10000
E = 320000
IN_DIM = 128
HID_DIM = 128
OUT_DIM = 16
NODE = 500

NC = 2
NS = 16
NW = NC * NS
EW = E // NW
C = 250
NCHUNK = EW // C
CS = 625
NCHUNKS = EW // CS
NPAD = 10240
RPT = NPAD // NS

_H2 = HID_DIM // 2
_f32 = jnp.float32
_SDS = jax.ShapeDtypeStruct
_SC_PARAMS = pltpu.CompilerParams(use_tc_tiling_on_sc=False)


def _make_mesh():
    return plsc.VectorSubcoreMesh(core_axis_name="c", subcore_axis_name="s",
                                  num_cores=NC, num_subcores=NS)


DW = 16


@functools.cache
def _make_deg_kernel():
    return functools.partial(
        pl.kernel,
        out_type=(_SDS((NC, NPAD, DW), _f32), _SDS((NC, NPAD, DW), _f32)),
        mesh=_make_mesh(),
        scratch_types=[
            pltpu.VMEM((NCHUNKS, CS), jnp.int32),
            pltpu.VMEM((NCHUNKS, CS), jnp.int32),
            pltpu.VMEM((CS, DW), _f32),
            pltpu.VMEM_SHARED((NPAD, DW), _f32),
            pltpu.VMEM_SHARED((NPAD, DW), _f32),
        ],
        compiler_params=_SC_PARAMS,
    )(_deg_body)


def _deg_body(edges_hbm, ones_hbm, zeros_hbm, dout_hbm, din_hbm,
              srcv, dstv, onesv, hout, hin):
    c = lax.axis_index("c")
    s = lax.axis_index("s")
    wid = c * NS + s
    pltpu.sync_copy(edges_hbm.at[0, wid], srcv)
    pltpu.sync_copy(edges_hbm.at[1, wid], dstv)
    pltpu.sync_copy(ones_hbm, onesv)
    r0 = s * RPT
    pltpu.sync_copy(zeros_hbm.at[pl.ds(r0, RPT)], hout.at[pl.ds(r0, RPT)])
    pltpu.sync_copy(zeros_hbm.at[pl.ds(r0, RPT)], hin.at[pl.ds(r0, RPT)])
    plsc.subcore_barrier()

    @pl.loop(0, NCHUNKS)
    def _(j):
        pltpu.sync_copy(onesv, hout.at[srcv.at[j]], add=True)
        pltpu.sync_copy(onesv, hin.at[dstv.at[j]], add=True)

    plsc.subcore_barrier()
    pltpu.sync_copy(hout.at[pl.ds(r0, RPT)], dout_hbm.at[c, pl.ds(r0, RPT)])
    pltpu.sync_copy(hin.at[pl.ds(r0, RPT)], din_hbm.at[c, pl.ds(r0, RPT)])


def _agg_body(rows_hbm, edges_hbm, ain_hbm, b2_hbm, zeros_hbm, out_hbm,
              srcv, dstv, buf0, buf1, acc, rbuf, ainv, b2v, sem0, sem1):
    c = lax.axis_index("c")
    s = lax.axis_index("s")
    wid = c * NS + s
    pltpu.sync_copy(edges_hbm.at[0, wid], srcv)
    pltpu.sync_copy(edges_hbm.at[1, wid], dstv)
    r0 = s * RPT
    pltpu.sync_copy(zeros_hbm.at[pl.ds(r0, RPT)], acc.at[pl.ds(r0, RPT)])
    plsc.subcore_barrier()

    def start(j, buf, sem):
        pltpu.async_copy(rows_hbm.at[srcv.at[j]], buf, sem)

    def finish(j, buf, sem):
        pltpu.make_async_copy(rows_hbm.at[srcv.at[j]], buf, sem).wait()

    def scatadd(j, buf):
        pltpu.sync_copy(buf, acc.at[dstv.at[j]], add=True)

    nch = srcv.shape[0]
    start(0, buf0, sem0)

    @pl.loop(0, nch - 2, step=2)
    def _(j):
        start(j + 1, buf1, sem1)
        finish(j, buf0, sem0)
        scatadd(j, buf0)
        start(j + 2, buf0, sem0)
        finish(j + 1, buf1, sem1)
        scatadd(j + 1, buf1)

    start(nch - 1, buf1, sem1)
    finish(nch - 2, buf0, sem0)
    scatadd(nch - 2, buf0)
    finish(nch - 1, buf1, sem1)
    scatadd(nch - 1, buf1)

    plsc.subcore_barrier()
    pltpu.sync_copy(acc.at[pl.ds(r0, RPT)], rbuf)
    pltpu.sync_copy(ain_hbm.at[pl.ds(r0, RPT)], ainv)
    pltpu.sync_copy(b2_hbm, b2v)
    fac = jnp.where(c == 0, 1.0, 0.0)
    b2eff = b2v[0] * fac

    @pl.loop(0, RPT)
    def _(r):
        rbuf[r] = rbuf[r] * ainv[r] + b2eff

    pltpu.sync_copy(rbuf, out_hbm.at[c, pl.ds(r0, RPT)])


def _agg_chunks(D):
    return (CS, NCHUNKS) if D <= 16 else (C, NCHUNK)


def _agg_scratch(D):
    c_, n_ = _agg_chunks(D)
    return [
        pltpu.VMEM((n_, c_), jnp.int32),
        pltpu.VMEM((n_, c_), jnp.int32),
        pltpu.VMEM((c_, D), _f32),
        pltpu.VMEM((c_, D), _f32),
        pltpu.VMEM_SHARED((NPAD, D), _f32),
        pltpu.VMEM((RPT, D), _f32),
        pltpu.VMEM((RPT, DW), _f32),
        pltpu.VMEM((1, D), _f32),
        pltpu.SemaphoreType.DMA,
        pltpu.SemaphoreType.DMA,
    ]


def _agg2_body(rowsa_hbm, rowsb_hbm, edges_hbm, zeros_hbm, out_hbm,
               srcv, dstv, buf0, buf1, acc, sem0, sem1):
    c = lax.axis_index("c")
    s = lax.axis_index("s")
    wid = c * NS + s
    pltpu.sync_copy(edges_hbm.at[0, wid], srcv)
    pltpu.sync_copy(edges_hbm.at[1, wid], dstv)
    r0 = s * RPT

    for half, rows_hbm in enumerate((rowsa_hbm, rowsb_hbm)):
        col0 = half * _H2

        def start(j, buf, sem, rows_hbm=rows_hbm):
            pltpu.async_copy(rows_hbm.at[srcv.at[j]], buf, sem)

        def finish(j, buf, sem, rows_hbm=rows_hbm):
            pltpu.make_async_copy(rows_hbm.at[srcv.at[j]], buf, sem).wait()

        def scatadd(j, buf):
            pltpu.sync_copy(buf, acc.at[dstv.at[j]], add=True)

        pltpu.sync_copy(zeros_hbm.at[pl.ds(r0, RPT)], acc.at[pl.ds(r0, RPT)])
        plsc.subcore_barrier()

        start(0, buf0, sem0)

        @pl.loop(0, NCHUNK - 2, step=2)
        def _(j):
            start(j + 1, buf1, sem1)
            finish(j, buf0, sem0)
            scatadd(j, buf0)
            start(j + 2, buf0, sem0)
            finish(j + 1, buf1, sem1)
            scatadd(j + 1, buf1)

        start(NCHUNK - 1, buf1, sem1)
        finish(NCHUNK - 2, buf0, sem0)
        scatadd(NCHUNK - 2, buf0)
        finish(NCHUNK - 1, buf1, sem1)
        scatadd(NCHUNK - 1, buf1)

        plsc.subcore_barrier()
        pltpu.sync_copy(acc.at[pl.ds(r0, RPT)],
                        out_hbm.at[c, pl.ds(r0, RPT), pl.ds(col0, _H2)])


@functools.cache
def _make_agg2():
    return functools.partial(
        pl.kernel,
        out_type=_SDS((NC, NPAD, HID_DIM), _f32),
        mesh=_make_mesh(),
        scratch_types=[
            pltpu.VMEM((NCHUNK, C), jnp.int32),
            pltpu.VMEM((NCHUNK, C), jnp.int32),
            pltpu.VMEM((C, _H2), _f32),
            pltpu.VMEM((C, _H2), _f32),
            pltpu.VMEM_SHARED((NPAD, _H2), _f32),
            pltpu.SemaphoreType.DMA,
            pltpu.SemaphoreType.DMA,
        ],
        compiler_params=_SC_PARAMS,
    )(_agg2_body)


@functools.cache
def _make_agg(D):
    return functools.partial(
        pl.kernel,
        out_type=_SDS((NC, NPAD, D), _f32),
        mesh=_make_mesh(),
        scratch_types=_agg_scratch(D),
        compiler_params=_SC_PARAMS,
    )(_agg_body)


_R = 1000
_G = N // _R


def _tc1_body(x_ref, f_ref, w_ref, dop_ref, dip_ref,
              hwsa_ref, hwsb_ref, ain_ref, aout_ref):
    i = pl.program_id(0)
    tail = jnp.concatenate([x_ref[:_R - NODE], f_ref[...]], axis=0)
    feat = jnp.where(i == _G - 1, tail, x_ref[...])
    a_out = lax.rsqrt(jnp.maximum(dop_ref[:, :1], 1.0))
    a_in = lax.rsqrt(jnp.maximum(dip_ref[:, :1], 1.0))
    hw = jnp.dot(feat, w_ref[...], preferred_element_type=_f32)
    hws = hw * a_out
    hwsa_ref[...] = hws[:, :_H2]
    hwsb_ref[...] = hws[:, _H2:]
    ain_ref[...] = jnp.broadcast_to(a_in, (_R, DW))
    aout_ref[...] = a_out


def _tc1(x, feature, W1, dout_p, din_p):
    return pl.pallas_call(
        _tc1_body,
        grid=(_G,),
        in_specs=[
            pl.BlockSpec((_R, IN_DIM), lambda i: (i, 0)),
            pl.BlockSpec((NODE, IN_DIM), lambda i: (0, 0)),
            pl.BlockSpec((IN_DIM, HID_DIM), lambda i: (0, 0)),
            pl.BlockSpec((_R, DW), lambda i: (i, 0)),
            pl.BlockSpec((_R, DW), lambda i: (i, 0)),
        ],
        out_specs=[
            pl.BlockSpec((_R, _H2), lambda i: (i, 0)),
            pl.BlockSpec((_R, _H2), lambda i: (i, 0)),
            pl.BlockSpec((_R, DW), lambda i: (i, 0)),
            pl.BlockSpec((_R, 1), lambda i: (i, 0)),
        ],
        out_shape=[_SDS((N, _H2), _f32), _SDS((N, _H2), _f32),
                   _SDS((NPAD, DW), _f32), _SDS((NPAD, 1), _f32)],
    )(x, feature, W1, dout_p, din_p)


def _tc2_body(agg_ref, ain_ref, b1_ref, w2_ref, aout_ref, out_ref):
    h = jnp.maximum((agg_ref[0] + agg_ref[1]) * ain_ref[:, :1] + b1_ref[...],
                    0.0)
    hw2 = jnp.dot(h, w2_ref[...], preferred_element_type=_f32)
    out_ref[...] = hw2 * aout_ref[...]


def _tc2(agg_p, a_in, b1, W2, a_out):
    return pl.pallas_call(
        _tc2_body,
        grid=(_G,),
        in_specs=[
            pl.BlockSpec((NC, _R, HID_DIM), lambda i: (0, i, 0)),
            pl.BlockSpec((_R, DW), lambda i: (i, 0)),
            pl.BlockSpec((1, HID_DIM), lambda i: (0, 0)),
            pl.BlockSpec((HID_DIM, OUT_DIM), lambda i: (0, 0)),
            pl.BlockSpec((_R, 1), lambda i: (i, 0)),
        ],
        out_specs=pl.BlockSpec((_R, OUT_DIM), lambda i: (i, 0)),
        out_shape=_SDS((N, OUT_DIM), _f32),
    )(agg_p, a_in, b1, W2, a_out)


def kernel(x, edge_index, feature, W1, b1, W2, b2):
    e_c = edge_index.reshape(2, NW, NCHUNK, C)
    e_cs = edge_index.reshape(2, NW, NCHUNKS, CS)
    ones = jnp.ones((CS, DW), _f32)
    zeros64 = jnp.zeros((NPAD, _H2), _f32)
    zeros16 = jnp.zeros((NPAD, OUT_DIM), _f32)

    dout_p, din_p = _make_deg_kernel()(e_cs, ones, zeros16)
    hwsa, hwsb, a_in, a_out = _tc1(x, feature, W1, dout_p[0] + dout_p[1],
                                   din_p[0] + din_p[1])
    agg1_p = _make_agg2()(hwsa, hwsb, e_c, zeros64)
    hw2s = _tc2(agg1_p, a_in, b1.reshape(1, HID_DIM), W2, a_out)
    agg2_p = _make_agg(OUT_DIM)(hw2s, e_cs, a_in, b2.reshape(1, OUT_DIM),
                                zeros16)
    return agg2_p[0, :N] + agg2_p[1, :N]


# speedup vs baseline: 1.0357x; 1.0357x over previous
import functools

import jax
import jax.numpy as jnp
from jax import lax
from jax.experimental import pallas as pl
from jax.experimental.pallas import tpu as pltpu
from jax.experimental.pallas import tpu_sc as plsc

N = 10000
E = 320000
IN_DIM = 128
HID_DIM = 128
OUT_DIM = 16
NODE = 500

NC = 2
NS = 16
NW = NC * NS
EW = E // NW
C = 250
NCHUNK = EW // C
CS = 625
NCHUNKS = EW // CS
NPAD = 10240
RPT = NPAD // NS

_H2 = HID_DIM // 2
_f32 = jnp.float32
_SDS = jax.ShapeDtypeStruct
_SC_PARAMS = pltpu.CompilerParams(use_tc_tiling_on_sc=False)


def _make_mesh():
    return plsc.VectorSubcoreMesh(core_axis_name="c", subcore_axis_name="s",
                                  num_cores=NC, num_subcores=NS)


DW = 16


@functools.cache
def _make_deg_kernel():
    return functools.partial(
        pl.kernel,
        out_type=(_SDS((NC, NPAD, DW), _f32), _SDS((NC, NPAD, DW), _f32)),
        mesh=_make_mesh(),
        scratch_types=[
            pltpu.VMEM((NCHUNKS, CS), jnp.int32),
            pltpu.VMEM((NCHUNKS, CS), jnp.int32),
            pltpu.VMEM((CS, DW), _f32),
            pltpu.VMEM_SHARED((NPAD, DW), _f32),
            pltpu.VMEM_SHARED((NPAD, DW), _f32),
        ],
        compiler_params=_SC_PARAMS,
    )(_deg_body)


def _deg_body(edges_hbm, ones_hbm, zeros_hbm, dout_hbm, din_hbm,
              srcv, dstv, onesv, hout, hin):
    c = lax.axis_index("c")
    s = lax.axis_index("s")
    wid = c * NS + s
    pltpu.sync_copy(edges_hbm.at[0, wid], srcv)
    pltpu.sync_copy(edges_hbm.at[1, wid], dstv)
    pltpu.sync_copy(ones_hbm, onesv)
    r0 = s * RPT
    pltpu.sync_copy(zeros_hbm.at[pl.ds(r0, RPT)], hout.at[pl.ds(r0, RPT)])
    pltpu.sync_copy(zeros_hbm.at[pl.ds(r0, RPT)], hin.at[pl.ds(r0, RPT)])
    plsc.subcore_barrier()

    @pl.loop(0, NCHUNKS)
    def _(j):
        pltpu.sync_copy(onesv, hout.at[srcv.at[j]], add=True)
        pltpu.sync_copy(onesv, hin.at[dstv.at[j]], add=True)

    plsc.subcore_barrier()
    pltpu.sync_copy(hout.at[pl.ds(r0, RPT)], dout_hbm.at[c, pl.ds(r0, RPT)])
    pltpu.sync_copy(hin.at[pl.ds(r0, RPT)], din_hbm.at[c, pl.ds(r0, RPT)])


def _agg_body(rows_hbm, edges_hbm, ain_hbm, b2_hbm, zeros_hbm, out_hbm,
              srcv, dstv, buf0, buf1, acc, rbuf, ainv, b2v, sem0, sem1):
    c = lax.axis_index("c")
    s = lax.axis_index("s")
    wid = c * NS + s
    pltpu.sync_copy(edges_hbm.at[0, wid], srcv)
    pltpu.sync_copy(edges_hbm.at[1, wid], dstv)
    r0 = s * RPT
    pltpu.sync_copy(zeros_hbm.at[pl.ds(r0, RPT)], acc.at[pl.ds(r0, RPT)])
    plsc.subcore_barrier()

    def start(j, buf, sem):
        pltpu.async_copy(rows_hbm.at[srcv.at[j]], buf, sem)

    def finish(j, buf, sem):
        pltpu.make_async_copy(rows_hbm.at[srcv.at[j]], buf, sem).wait()

    def scatadd(j, buf):
        pltpu.sync_copy(buf, acc.at[dstv.at[j]], add=True)

    nch = srcv.shape[0]
    start(0, buf0, sem0)

    @pl.loop(0, nch - 2, step=2)
    def _(j):
        start(j + 1, buf1, sem1)
        finish(j, buf0, sem0)
        scatadd(j, buf0)
        start(j + 2, buf0, sem0)
        finish(j + 1, buf1, sem1)
        scatadd(j + 1, buf1)

    start(nch - 1, buf1, sem1)
    finish(nch - 2, buf0, sem0)
    scatadd(nch - 2, buf0)
    finish(nch - 1, buf1, sem1)
    scatadd(nch - 1, buf1)

    plsc.subcore_barrier()
    pltpu.sync_copy(acc.at[pl.ds(r0, RPT)], rbuf)
    pltpu.sync_copy(ain_hbm.at[pl.ds(r0, RPT)], ainv)
    pltpu.sync_copy(b2_hbm, b2v)
    fac = jnp.where(c == 0, 1.0, 0.0)
    b2eff = b2v[0] * fac

    @pl.loop(0, RPT)
    def _(r):
        rbuf[r] = rbuf[r] * ainv[r] + b2eff

    pltpu.sync_copy(rbuf, out_hbm.at[c, pl.ds(r0, RPT)])


def _agg_chunks(D):
    return (CS, NCHUNKS) if D <= 16 else (C, NCHUNK)


def _agg_scratch(D):
    c_, n_ = _agg_chunks(D)
    return [
        pltpu.VMEM((n_, c_), jnp.int32),
        pltpu.VMEM((n_, c_), jnp.int32),
        pltpu.VMEM((c_, D), _f32),
        pltpu.VMEM((c_, D), _f32),
        pltpu.VMEM_SHARED((NPAD, D), _f32),
        pltpu.VMEM((RPT, D), _f32),
        pltpu.VMEM((RPT, DW), _f32),
        pltpu.VMEM((1, D), _f32),
        pltpu.SemaphoreType.DMA,
        pltpu.SemaphoreType.DMA,
    ]


def _agg2_body(rowsa_hbm, rowsb_hbm, edges_hbm, zeros_hbm, out_hbm,
               srcv, dstv, buf0, buf1, acc, sem0, sem1):
    c = lax.axis_index("c")
    s = lax.axis_index("s")
    wid = c * NS + s
    pltpu.sync_copy(edges_hbm.at[0, wid], srcv)
    pltpu.sync_copy(edges_hbm.at[1, wid], dstv)
    r0 = s * RPT

    for half, rows_hbm in enumerate((rowsa_hbm, rowsb_hbm)):
        col0 = half * _H2

        def start(j, buf, sem, rows_hbm=rows_hbm):
            pltpu.async_copy(rows_hbm.at[srcv.at[j]], buf, sem)

        def finish(j, buf, sem, rows_hbm=rows_hbm):
            pltpu.make_async_copy(rows_hbm.at[srcv.at[j]], buf, sem).wait()

        def scatadd(j, buf):
            pltpu.sync_copy(buf, acc.at[dstv.at[j]], add=True)

        pltpu.sync_copy(zeros_hbm.at[pl.ds(r0, RPT)], acc.at[pl.ds(r0, RPT)])
        plsc.subcore_barrier()

        start(0, buf0, sem0)

        @pl.loop(0, NCHUNK - 2, step=2)
        def _(j):
            start(j + 1, buf1, sem1)
            finish(j, buf0, sem0)
            scatadd(j, buf0)
            start(j + 2, buf0, sem0)
            finish(j + 1, buf1, sem1)
            scatadd(j + 1, buf1)

        start(NCHUNK - 1, buf1, sem1)
        finish(NCHUNK - 2, buf0, sem0)
        scatadd(NCHUNK - 2, buf0)
        finish(NCHUNK - 1, buf1, sem1)
        scatadd(NCHUNK - 1, buf1)

        plsc.subcore_barrier()
        pltpu.sync_copy(acc.at[pl.ds(r0, RPT)],
                        out_hbm.at[c, pl.ds(r0, RPT), pl.ds(col0, _H2)])


@functools.cache
def _make_agg2():
    return functools.partial(
        pl.kernel,
        out_type=_SDS((NC, NPAD, HID_DIM), _f32),
        mesh=_make_mesh(),
        scratch_types=[
            pltpu.VMEM((NCHUNK, C), jnp.int32),
            pltpu.VMEM((NCHUNK, C), jnp.int32),
            pltpu.VMEM((C, _H2), _f32),
            pltpu.VMEM((C, _H2), _f32),
            pltpu.VMEM_SHARED((NPAD, _H2), _f32),
            pltpu.SemaphoreType.DMA,
            pltpu.SemaphoreType.DMA,
        ],
        compiler_params=_SC_PARAMS,
    )(_agg2_body)


@functools.cache
def _make_agg(D):
    return functools.partial(
        pl.kernel,
        out_type=_SDS((NC, NPAD, D), _f32),
        mesh=_make_mesh(),
        scratch_types=_agg_scratch(D),
        compiler_params=_SC_PARAMS,
    )(_agg_body)


_R = 1000
_G = N // _R


def _tc1_body(x_ref, f_ref, w_ref, dop_ref, dip_ref,
              hwsa_ref, hwsb_ref, ain_ref, aout_ref):
    i = pl.program_id(0)
    tail = jnp.concatenate([x_ref[:_R - NODE], f_ref[...]], axis=0)
    feat = jnp.where(i == _G - 1, tail, x_ref[...])
    a_out = lax.rsqrt(jnp.maximum((dop_ref[0] + dop_ref[1])[:, :1], 1.0))
    a_in = lax.rsqrt(jnp.maximum((dip_ref[0] + dip_ref[1])[:, :1], 1.0))
    hw = jnp.dot(feat, w_ref[...], preferred_element_type=_f32)
    hws = hw * a_out
    hwsa_ref[...] = hws[:, :_H2]
    hwsb_ref[...] = hws[:, _H2:]
    ain_ref[...] = jnp.broadcast_to(a_in, (_R, DW))
    aout_ref[...] = a_out


def _tc1(x, feature, W1, dout_p, din_p):
    return pl.pallas_call(
        _tc1_body,
        grid=(_G,),
        in_specs=[
            pl.BlockSpec((_R, IN_DIM), lambda i: (i, 0)),
            pl.BlockSpec((NODE, IN_DIM), lambda i: (0, 0)),
            pl.BlockSpec((IN_DIM, HID_DIM), lambda i: (0, 0)),
            pl.BlockSpec((NC, _R, DW), lambda i: (0, i, 0)),
            pl.BlockSpec((NC, _R, DW), lambda i: (0, i, 0)),
        ],
        out_specs=[
            pl.BlockSpec((_R, _H2), lambda i: (i, 0)),
            pl.BlockSpec((_R, _H2), lambda i: (i, 0)),
            pl.BlockSpec((_R, DW), lambda i: (i, 0)),
            pl.BlockSpec((_R, 1), lambda i: (i, 0)),
        ],
        out_shape=[_SDS((N, _H2), _f32), _SDS((N, _H2), _f32),
                   _SDS((NPAD, DW), _f32), _SDS((NPAD, 1), _f32)],
    )(x, feature, W1, dout_p, din_p)


def _tc2_body(agg_ref, ain_ref, b1_ref, w2_ref, aout_ref, out_ref):
    h = jnp.maximum((agg_ref[0] + agg_ref[1]) * ain_ref[:, :1] + b1_ref[...],
                    0.0)
    hw2 = jnp.dot(h, w2_ref[...], preferred_element_type=_f32)
    out_ref[...] = hw2 * aout_ref[...]


def _tc2(agg_p, a_in, b1, W2, a_out):
    return pl.pallas_call(
        _tc2_body,
        grid=(_G,),
        in_specs=[
            pl.BlockSpec((NC, _R, HID_DIM), lambda i: (0, i, 0)),
            pl.BlockSpec((_R, DW), lambda i: (i, 0)),
            pl.BlockSpec((1, HID_DIM), lambda i: (0, 0)),
            pl.BlockSpec((HID_DIM, OUT_DIM), lambda i: (0, 0)),
            pl.BlockSpec((_R, 1), lambda i: (i, 0)),
        ],
        out_specs=pl.BlockSpec((_R, OUT_DIM), lambda i: (i, 0)),
        out_shape=_SDS((N, OUT_DIM), _f32),
    )(agg_p, a_in, b1, W2, a_out)


def kernel(x, edge_index, feature, W1, b1, W2, b2):
    e_c = edge_index.reshape(2, NW, NCHUNK, C)
    e_cs = edge_index.reshape(2, NW, NCHUNKS, CS)
    ones = jnp.ones((CS, DW), _f32)
    zeros64 = jnp.zeros((NPAD, _H2), _f32)
    zeros16 = jnp.zeros((NPAD, OUT_DIM), _f32)

    dout_p, din_p = _make_deg_kernel()(e_cs, ones, zeros16)
    hwsa, hwsb, a_in, a_out = _tc1(x, feature, W1, dout_p, din_p)
    agg1_p = _make_agg2()(hwsa, hwsb, e_c, zeros64)
    hw2s = _tc2(agg1_p, a_in, b1.reshape(1, HID_DIM), W2, a_out)
    agg2_p = _make_agg(OUT_DIM)(hw2s, e_cs, a_in, b2.reshape(1, OUT_DIM),
                                zeros16)
    return agg2_p[0, :N] + agg2_p[1, :N]
